# Initial kernel scaffold; baseline (speedup 1.0000x reference)
import jax
import jax.numpy as jnp
from jax.experimental import pallas as pl


def kernel(x_nchw, w_oihw, b, gamma, beta):
    raise NotImplementedError("write your pallas kernel here")


# speedup vs baseline: 1.0051x; 1.0051x over previous
import functools

import jax
import jax.numpy as jnp
from jax import lax
from jax.experimental import pallas as pl
from jax.experimental.pallas import tpu as pltpu

EPS = 1e-5
BF16 = jnp.bfloat16
F32 = jnp.float32


def _make_conv_stats_kernel(h):
    def conv_stats_kernel(x_ref, w_ref, conv_ref, stats_ref):
        bpb, hp, wpc = x_ref.shape
        rows = [x_ref[:, kh:kh + h, :].reshape(bpb * h, wpc) for kh in range(3)]
        lhs = jnp.concatenate(rows, axis=-1)
        acc = jnp.dot(lhs, w_ref[...], preferred_element_type=F32)
        conv_ref[...] = acc.astype(BF16)
        s = jnp.sum(acc, axis=0, keepdims=True)
        sq = jnp.sum(acc * acc, axis=0, keepdims=True)
        pad = jnp.zeros((stats_ref.shape[0] - 2, stats_ref.shape[1]), F32)
        stats_ref[...] = jnp.concatenate([s, sq, pad], axis=0)
    return conv_stats_kernel


def _bn_relu_kernel(c_ref, scale_ref, shift_ref, o_ref):
    y = jnp.maximum(c_ref[...].astype(F32) * scale_ref[...] + shift_ref[...],
                    0.0)
    o_ref[...] = y.astype(BF16)


def _pick_bpb(n, h):
    for cand in (64, 32, 16, 8, 4, 2):
        if n % cand == 0 and (cand * h) % 8 == 0:
            return cand
    return n


@jax.jit
def _chess_conv_block(x_nchw, w_oihw, gamma, beta):
    n, cin, h, w = x_nchw.shape
    cout = w_oihw.shape[0]

    x_nhwc = jnp.transpose(x_nchw, (0, 2, 3, 1))
    x_pad = jnp.pad(x_nhwc, ((0, 0), (1, 1), (1, 1), (0, 0))).astype(BF16)
    x_flat = x_pad.reshape(n, h + 2, (w + 2) * cin)

    w_hwio = jnp.transpose(w_oihw, (2, 3, 1, 0)).astype(F32)
    eye_w = jnp.eye(w, dtype=F32)
    bands = []
    for kh in range(3):
        band = jnp.zeros(((w + 2) * cin, w * cout), F32)
        for kw in range(3):
            blockdiag = jnp.kron(eye_w, w_hwio[kh, kw])
            band = band + jnp.pad(blockdiag,
                                  ((kw * cin, (2 - kw) * cin), (0, 0)))
        bands.append(band)
    big_w = jnp.concatenate(bands, axis=0).astype(BF16)

    bpb = _pick_bpb(n, h)
    nb = n // bpb
    rows_blk = bpb * h
    cw = cout * w

    cparams = pltpu.CompilerParams(
        dimension_semantics=("parallel",),
        vmem_limit_bytes=64 * 1024 * 1024)

    conv_flops = 2 * n * h * (3 * (w + 2) * cin) * cw
    conv_bytes = 2 * (x_flat.size + big_w.size + n * h * cw) + 4 * nb * 8 * cw
    conv2d, stats = pl.pallas_call(
        _make_conv_stats_kernel(h),
        grid=(nb,),
        in_specs=(
            pl.BlockSpec((bpb, h + 2, (w + 2) * cin), lambda i: (i, 0, 0)),
            pl.BlockSpec((3 * (w + 2) * cin, cw), lambda i: (0, 0)),
        ),
        out_specs=(
            pl.BlockSpec((rows_blk, cw), lambda i: (i, 0)),
            pl.BlockSpec((8, cw), lambda i: (i, 0)),
        ),
        out_shape=(
            jax.ShapeDtypeStruct((n * h, cw), BF16),
            jax.ShapeDtypeStruct((nb * 8, cw), F32),
        ),
        compiler_params=cparams,
        cost_estimate=pl.CostEstimate(flops=conv_flops, transcendentals=0,
                                      bytes_accessed=conv_bytes),
    )(x_flat, big_w)

    m_total = n * h * w
    st = stats.reshape(nb, 8, cw)
    ch_sum = jnp.sum(st[:, 0, :], axis=0).reshape(w, cout).sum(axis=0)
    ch_sq = jnp.sum(st[:, 1, :], axis=0).reshape(w, cout).sum(axis=0)
    mean = ch_sum / m_total
    var = jnp.maximum(ch_sq / m_total - mean * mean, 0.0)
    inv_std = lax.rsqrt(var + EPS)
    scale = gamma.astype(F32) * inv_std
    shift = beta.astype(F32) - mean * scale
    scale_row = jnp.tile(scale, w).reshape(1, cw)
    shift_row = jnp.tile(shift, w).reshape(1, cw)

    out2d = pl.pallas_call(
        _bn_relu_kernel,
        grid=(nb,),
        in_specs=(
            pl.BlockSpec((rows_blk, cw), lambda i: (i, 0)),
            pl.BlockSpec((1, cw), lambda i: (0, 0)),
            pl.BlockSpec((1, cw), lambda i: (0, 0)),
        ),
        out_specs=pl.BlockSpec((rows_blk, cw), lambda i: (i, 0)),
        out_shape=jax.ShapeDtypeStruct((n * h, cw), BF16),
        compiler_params=cparams,
        cost_estimate=pl.CostEstimate(flops=3 * n * h * cw, transcendentals=0,
                                      bytes_accessed=4 * n * h * cw),
    )(conv2d, scale_row, shift_row)

    return jnp.transpose(out2d.reshape(n, h, w, cout),
                         (0, 3, 1, 2)).astype(F32)


def kernel(x_nchw, w_oihw, b, gamma, beta):
    del b
    return _chess_conv_block(x_nchw, w_oihw, gamma, beta)
